# baseline (speedup 1.0000x reference)
import functools

import jax
import jax.numpy as jnp
from jax import lax
from jax.experimental import pallas as pl
from jax.experimental.pallas import tpu as pltpu
from jax.experimental.pallas import tpu_sc as plsc

B = 16384
L = 50
D = 64
NC = 2
NS = 16
NW = NC * NS
SPT = B // NW
S = 8
NCHUNK = SPT // S
NBUF = 4
NLANE = 16
NGRP = D // (2 * NLANE)


def _body(spans_hbm, we_hbm, out_hbm, idx_all, rows, out_v, *sems):
    wid = lax.axis_index("s") * NC + lax.axis_index("c")
    base = wid * SPT

    pltpu.sync_copy(spans_hbm.at[pl.ds(base * L, SPT * L)], idx_all)

    def gather_start(c, b):
        pltpu.async_copy(
            we_hbm.at[idx_all.at[pl.ds(c * (S * L), S * L)]], rows.at[b], sems[b]
        )

    def gather_wait(c, b):
        pltpu.make_async_copy(
            we_hbm.at[idx_all.at[pl.ds(c * (S * L), S * L)]], rows.at[b], sems[b]
        ).wait()

    lanes = lax.iota(jnp.int32, NLANE)

    def pool_chunk(c, b):
        for s in range(S):
            zeros = jnp.zeros((NLANE,), jnp.float32)
            accs = (zeros,) * (2 * NGRP)

            def _acc(j, accs, s=s, b=b):
                new = []
                for g in range(NGRP):
                    v = plsc.bitcast(
                        rows[b, s * L + j, pl.ds(g * 2 * NLANE, 2 * NLANE)],
                        jnp.int32,
                    )
                    lo = plsc.bitcast(lax.shift_left(v, 16), jnp.float32)
                    hi = plsc.bitcast(
                        lax.bitwise_and(v, jnp.int32(-65536)), jnp.float32
                    )
                    new.append(accs[2 * g] + lo)
                    new.append(accs[2 * g + 1] + hi)
                return tuple(new)

            accs = lax.fori_loop(0, L, _acc, accs, unroll=10)
            row_idx = jnp.full((NLANE,), s, jnp.int32)
            for g in range(NGRP):
                cols = g * 2 * NLANE + 2 * lanes
                plsc.store_scatter(out_v, [row_idx, cols], accs[2 * g])
                plsc.store_scatter(out_v, [row_idx, cols + 1], accs[2 * g + 1])
        pltpu.sync_copy(out_v, out_hbm.at[pl.ds(base + c * S, S), :])

    gather_start(0, 0)
    gather_start(1, 1)

    @pl.loop(0, NCHUNK - NBUF, step=NBUF)
    def _ring(c0):
        for b in range(NBUF):
            c = c0 + b
            gather_wait(c, b)
            gather_start(c + 2, (b + 2) % NBUF)
            pool_chunk(c, b)

    for b in range(NBUF):
        c = NCHUNK - NBUF + b
        gather_wait(c, b)
        if b < 2:
            gather_start(c + 2, (b + 2) % NBUF)
        pool_chunk(c, b)


@jax.jit
def kernel(spans, we):
    spans_flat = spans.reshape(-1).astype(jnp.int32)
    we_bf = we.astype(jnp.bfloat16)
    mesh = plsc.VectorSubcoreMesh(
        core_axis_name="c", subcore_axis_name="s", num_cores=NC, num_subcores=NS
    )
    f = pl.kernel(
        _body,
        out_type=jax.ShapeDtypeStruct((B, D), jnp.float32),
        mesh=mesh,
        scratch_types=[
            pltpu.VMEM((SPT * L,), jnp.int32),
            pltpu.VMEM((NBUF, S * L, D), jnp.bfloat16),
            pltpu.VMEM((S, D), jnp.float32),
        ] + [pltpu.SemaphoreType.DMA] * NBUF,
        compiler_params=pltpu.CompilerParams(
            use_tc_tiling_on_sc=False, needs_layout_passes=False
        ),
    )
    return f(spans_flat, we_bf)


# speedup vs baseline: 1.0072x; 1.0072x over previous
import functools

import jax
import jax.numpy as jnp
from jax import lax
from jax.experimental import pallas as pl
from jax.experimental.pallas import tpu as pltpu
from jax.experimental.pallas import tpu_sc as plsc

B = 16384
L = 50
D = 64
NC = 2
NS = 16
NW = NC * NS
SPT = B // NW
HSPT = SPT // 2
S = 8
NCHUNK = HSPT // S
NBUF = 4
NLANE = 16
NGRP = D // (2 * NLANE)


def _body(spans_hbm, we_hbm, out_hbm, sp_v, idx_half, rows, out_vt, *sems):
    wid = lax.axis_index("s") * NC + lax.axis_index("c")
    base = wid * SPT
    lanes = lax.iota(jnp.int32, NLANE)

    def gather_start(c, b):
        pltpu.async_copy(
            we_hbm.at[idx_half.at[pl.ds(c * (S * L), S * L)]], rows.at[b], sems[b]
        )

    def gather_wait(c, b):
        pltpu.make_async_copy(
            we_hbm.at[idx_half.at[pl.ds(c * (S * L), S * L)]], rows.at[b], sems[b]
        ).wait()

    def pool_chunk(c, b, h):
        @pl.loop(0, S)
        def _span(s):
            zeros = jnp.zeros((NLANE,), jnp.float32)
            accs = (zeros,) * (2 * NGRP)

            def _acc(j, accs, s=s, b=b):
                new = []
                for g in range(NGRP):
                    v = plsc.bitcast(
                        rows[b, s * L + j, pl.ds(g * 2 * NLANE, 2 * NLANE)],
                        jnp.int32,
                    )
                    lo = plsc.bitcast(lax.shift_left(v, 16), jnp.float32)
                    hi = plsc.bitcast(
                        lax.bitwise_and(v, jnp.int32(-65536)), jnp.float32
                    )
                    new.append(accs[2 * g] + lo)
                    new.append(accs[2 * g + 1] + hi)
                return tuple(new)

            accs = lax.fori_loop(0, L, _acc, accs, unroll=10)
            col = jnp.full((NLANE,), h * HSPT + c * S, jnp.int32) + s
            for g in range(NGRP):
                drows = g * 2 * NLANE + 2 * lanes
                plsc.store_scatter(out_vt, [drows, col], accs[2 * g])
                plsc.store_scatter(out_vt, [drows + 1, col], accs[2 * g + 1])

    for h in range(2):
        hb = base + h * HSPT
        pltpu.sync_copy(spans_hbm.at[:, pl.ds(hb, HSPT)], sp_v)

        @pl.loop(0, HSPT * L, step=NLANE, unroll=4)
        def _tr(w):
            wv = w + lanes
            s = wv // L
            j = wv - s * L
            idx_half[pl.ds(w, NLANE)] = plsc.load_gather(sp_v, [j, s])

        gather_start(0, 0)
        gather_start(1, 1)

        @pl.loop(0, NCHUNK, step=NBUF)
        def _ring(c0, h=h):
            for b in range(NBUF):
                c = c0 + b
                gather_wait(c, b)

                @pl.when(c + 2 < NCHUNK)
                def _():
                    gather_start(c + 2, (b + 2) % NBUF)

                pool_chunk(c, b, h)

    pltpu.sync_copy(out_vt, out_hbm.at[:, pl.ds(base, SPT)])


@jax.jit
def kernel(spans, we):
    spans_t = spans.T.astype(jnp.int32)
    we_bf = we.astype(jnp.bfloat16)
    mesh = plsc.VectorSubcoreMesh(
        core_axis_name="c", subcore_axis_name="s", num_cores=NC, num_subcores=NS
    )
    f = pl.kernel(
        _body,
        out_type=jax.ShapeDtypeStruct((D, B), jnp.float32),
        mesh=mesh,
        scratch_types=[
            pltpu.VMEM((L, HSPT), jnp.int32),
            pltpu.VMEM((HSPT * L,), jnp.int32),
            pltpu.VMEM((NBUF, S * L, D), jnp.bfloat16),
            pltpu.VMEM((D, SPT), jnp.float32),
        ] + [pltpu.SemaphoreType.DMA] * NBUF,
        compiler_params=pltpu.CompilerParams(
            use_tc_tiling_on_sc=False, needs_layout_passes=False
        ),
    )
    return f(spans_t, we_bf).T


# speedup vs baseline: 1.0080x; 1.0008x over previous
import functools

import jax
import jax.numpy as jnp
from jax import lax
from jax.experimental import pallas as pl
from jax.experimental.pallas import tpu as pltpu
from jax.experimental.pallas import tpu_sc as plsc

B = 16384
L = 50
D = 64
NC = 2
NS = 16
NW = NC * NS
SPT = B // NW
HSPT = SPT // 2
S = 8
NCHUNK = HSPT // S
NBUF = 4
NLANE = 16
NGRP = D // (2 * NLANE)


def _body(spans_hbm, we_hbm, out_hbm, sp_v, idx_half, rows, out_vt, *sems):
    wid = lax.axis_index("s") * NC + lax.axis_index("c")
    base = wid * SPT
    lanes = lax.iota(jnp.int32, NLANE)

    def gather_start(c, b):
        pltpu.async_copy(
            we_hbm.at[idx_half.at[pl.ds(c * (S * L), S * L)]], rows.at[b], sems[b]
        )

    def gather_wait(c, b):
        pltpu.make_async_copy(
            we_hbm.at[idx_half.at[pl.ds(c * (S * L), S * L)]], rows.at[b], sems[b]
        ).wait()

    def pool_chunk(c, b, h):
        @pl.loop(0, S)
        def _span(s):
            zeros = jnp.zeros((NLANE,), jnp.float32)
            accs = (zeros,) * (2 * NGRP)

            def _acc(j, accs, s=s, b=b):
                new = []
                for g in range(NGRP):
                    v = plsc.bitcast(
                        rows[b, s * L + j, pl.ds(g * 2 * NLANE, 2 * NLANE)],
                        jnp.int32,
                    )
                    lo = plsc.bitcast(lax.shift_left(v, 16), jnp.float32)
                    hi = plsc.bitcast(
                        lax.bitwise_and(v, jnp.int32(-65536)), jnp.float32
                    )
                    new.append(accs[2 * g] + lo)
                    new.append(accs[2 * g + 1] + hi)
                return tuple(new)

            accs = lax.fori_loop(0, L, _acc, accs, unroll=10)
            col = jnp.full((NLANE,), h * HSPT + c * S, jnp.int32) + s
            for g in range(NGRP):
                drows = g * 2 * NLANE + 2 * lanes
                plsc.store_scatter(out_vt, [drows, col], accs[2 * g])
                plsc.store_scatter(out_vt, [drows + 1, col], accs[2 * g + 1])

    for h in range(2):
        hb = base + h * HSPT
        pltpu.sync_copy(spans_hbm.at[pl.ds(0, L), pl.ds(hb, HSPT)], sp_v)

        @pl.loop(0, HSPT * L, step=NLANE, unroll=4)
        def _tr(w):
            wv = w + lanes
            s = wv // L
            j = wv - s * L
            idx_half[pl.ds(w, NLANE)] = plsc.load_gather(sp_v, [j, s])

        gather_start(0, 0)
        gather_start(1, 1)

        @pl.loop(0, NCHUNK, step=NBUF)
        def _ring(c0, h=h):
            for b in range(NBUF):
                c = c0 + b
                gather_wait(c, b)

                @pl.when(c + 2 < NCHUNK)
                def _():
                    gather_start(c + 2, (b + 2) % NBUF)

                pool_chunk(c, b, h)

    pltpu.sync_copy(out_vt, out_hbm.at[:, pl.ds(base, SPT)])


@jax.jit
def kernel(spans, we):
    spans_t = jnp.pad(spans.T.astype(jnp.int32), ((0, 6), (0, 0)))
    we_bf = we.astype(jnp.bfloat16)
    mesh = plsc.VectorSubcoreMesh(
        core_axis_name="c", subcore_axis_name="s", num_cores=NC, num_subcores=NS
    )
    f = pl.kernel(
        _body,
        out_type=jax.ShapeDtypeStruct((D, B), jnp.float32),
        mesh=mesh,
        scratch_types=[
            pltpu.VMEM((L, HSPT), jnp.int32),
            pltpu.VMEM((HSPT * L,), jnp.int32),
            pltpu.VMEM((NBUF, S * L, D), jnp.bfloat16),
            pltpu.VMEM((D, SPT), jnp.float32),
        ] + [pltpu.SemaphoreType.DMA] * NBUF,
        compiler_params=pltpu.CompilerParams(
            use_tc_tiling_on_sc=False, needs_layout_passes=False
        ),
    )
    return f(spans_t, we_bf).T


# speedup vs baseline: 1.0092x; 1.0012x over previous
import functools

import jax
import jax.numpy as jnp
from jax import lax
from jax.experimental import pallas as pl
from jax.experimental.pallas import tpu as pltpu
from jax.experimental.pallas import tpu_sc as plsc

B = 16384
L = 50
D = 64
NC = 2
NS = 16
NW = NC * NS
SPT = B // NW
S = 8
NCHUNK = SPT // S
NBUF = 4
NLANE = 16
NREG = D // NLANE


def _body(spans_hbm, we_hbm, out_hbm, idx_all, rows, out_v, *sems):
    wid = lax.axis_index("s") * NC + lax.axis_index("c")
    base = wid * SPT

    pltpu.sync_copy(spans_hbm.at[pl.ds(base * L, SPT * L)], idx_all)

    def gather_start(c, b):
        pltpu.async_copy(
            we_hbm.at[idx_all.at[pl.ds(c * (S * L), S * L)]], rows.at[b], sems[b]
        )

    def gather_wait(c, b):
        pltpu.make_async_copy(
            we_hbm.at[idx_all.at[pl.ds(c * (S * L), S * L)]], rows.at[b], sems[b]
        ).wait()

    def pool_chunk(c, b):
        for s in range(S):
            accs = tuple(jnp.zeros((NLANE,), jnp.float32) for _ in range(NREG))

            def _acc(j, accs, s=s, b=b):
                return tuple(
                    accs[d] + rows[b, s * L + j, pl.ds(d * NLANE, NLANE)]
                    for d in range(NREG)
                )

            accs = lax.fori_loop(0, L, _acc, accs, unroll=10)
            for d in range(NREG):
                out_v[s, pl.ds(d * NLANE, NLANE)] = accs[d]
        pltpu.sync_copy(out_v, out_hbm.at[pl.ds(base + c * S, S), :])

    gather_start(0, 0)
    gather_start(1, 1)

    @pl.loop(0, NCHUNK - NBUF, step=NBUF)
    def _ring(c0):
        for b in range(NBUF):
            c = c0 + b
            gather_wait(c, b)
            gather_start(c + 2, (b + 2) % NBUF)
            pool_chunk(c, b)

    for b in range(NBUF):
        c = NCHUNK - NBUF + b
        gather_wait(c, b)
        if b < 2:
            gather_start(c + 2, (b + 2) % NBUF)
        pool_chunk(c, b)


@jax.jit
def kernel(spans, we):
    spans_flat = spans.reshape(-1).astype(jnp.int32)
    mesh = plsc.VectorSubcoreMesh(
        core_axis_name="c", subcore_axis_name="s", num_cores=NC, num_subcores=NS
    )
    f = pl.kernel(
        _body,
        out_type=jax.ShapeDtypeStruct((B, D), jnp.float32),
        mesh=mesh,
        scratch_types=[
            pltpu.VMEM((SPT * L,), jnp.int32),
            pltpu.VMEM((NBUF, S * L, D), jnp.float32),
            pltpu.VMEM((S, D), jnp.float32),
        ] + [pltpu.SemaphoreType.DMA] * NBUF,
        compiler_params=pltpu.CompilerParams(use_tc_tiling_on_sc=False),
    )
    return f(spans_flat, we)
